# baseline (speedup 1.0000x reference)
import functools

import jax
import jax.numpy as jnp
from jax import lax
from jax.experimental import pallas as pl
from jax.experimental.pallas import tpu as pltpu
from jax.experimental.pallas import tpu_sc as plsc

B, C, H, W = 64, 768, 24, 24
HW = H * W
E = 8
NEG_INF = float("-inf")

B_TC = 48
B_SC = B - B_TC
W_PER_B = 2
HB = HW // W_PER_B
HBLK = 48
NBLK = HB // HBLK
CCH = C // 16

BB = 8
NSTEPS = B_TC // BB


def _pool_kernel(x_ref, out_ref):
    blk = x_ref[...]
    out_ref[...] = (jnp.max(blk, axis=1)
                    + jnp.sum(blk, axis=1) * (1.0 / HW))


def _route_kernel(ptc_ref, pmax_ref, psum_ref, w0_ref, b0_ref, w1_ref,
                  b1_ref, out_ref):
    psc = (jnp.max(pmax_ref[...], axis=0)
           + jnp.sum(psum_ref[...], axis=0) * (1.0 / HW))
    pooled = jnp.concatenate([ptc_ref[...], psc], axis=0)
    dn = (((1,), (1,)), ((), ()))
    h = jax.lax.dot_general(
        pooled, w0_ref[...], dn,
        preferred_element_type=jnp.float32) + b0_ref[...]
    h = jnp.where(h >= 0.0, h, 0.2 * h)
    z = jax.lax.dot_general(
        pooled, w1_ref[...], dn,
        preferred_element_type=jnp.float32) + b1_ref[...]
    noise = jnp.maximum(z, 0.0) + jnp.log1p(jnp.exp(-jnp.abs(z)))
    nmean = jnp.mean(noise, axis=1, keepdims=True)
    var = jnp.sum((noise - nmean) ** 2, axis=1, keepdims=True) / (E - 1)
    norm_noise = (noise - nmean) * jax.lax.rsqrt(var)
    scores = h + norm_noise
    ii = jax.lax.broadcasted_iota(jnp.int32, (B, E), 1)
    m1 = jnp.max(scores, axis=1, keepdims=True)
    i1 = jnp.min(jnp.where(scores == m1, ii, E), axis=1, keepdims=True)
    oh1 = ii == i1
    s2 = jnp.where(oh1, NEG_INF, scores)
    m2 = jnp.max(s2, axis=1, keepdims=True)
    i2 = jnp.min(jnp.where(s2 == m2, ii, E), axis=1, keepdims=True)
    mask = oh1 | (ii == i2)
    hm = jnp.where(mask, h, NEG_INF)
    mx = jnp.max(hm, axis=1, keepdims=True)
    e = jnp.where(mask, jnp.exp(h - mx), 0.0)
    gate = e / jnp.sum(e, axis=1, keepdims=True)
    out_ref[...] = gate.T


def _sc_pool(x_hbm, pmax_hbm, psum_hbm, xv0, xv1, accm, accs, sem0, sem1):
    nc = 2
    wid = lax.axis_index("s") * nc + lax.axis_index("c")
    b_local = wid // W_PER_B
    wslot = wid % W_PER_B
    b = B_TC + b_local
    h0 = wslot * HB
    bufs = (xv0, xv1)
    sems = (sem0, sem1)

    def dma(i, buf, sem):
        return pltpu.make_async_copy(
            x_hbm.at[b, pl.ds(h0 + i * HBLK, HBLK), :], buf, sem)

    def init_cc(cc, carry):
        accm[pl.ds(cc * 16, 16)] = jnp.full((16,), NEG_INF, jnp.float32)
        accs[pl.ds(cc * 16, 16)] = jnp.zeros((16,), jnp.float32)
        return carry
    lax.fori_loop(0, CCH, init_cc, 0)

    dma(0, bufs[0], sems[0]).start()
    for i in range(NBLK):
        cur = bufs[i % 2]
        dma(i, cur, sems[i % 2]).wait()
        if i + 1 < NBLK:
            dma(i + 1, bufs[(i + 1) % 2], sems[(i + 1) % 2]).start()

        def body(cc, carry):
            c16 = cc * 16
            v0 = cur[0, pl.ds(c16, 16)]
            v1 = cur[1, pl.ds(c16, 16)]
            v2 = cur[2, pl.ds(c16, 16)]
            v3 = cur[3, pl.ds(c16, 16)]
            am0, am1, am2, am3 = v0, v1, v2, v3
            as0, as1, as2, as3 = v0, v1, v2, v3
            for r in range(4, HBLK, 4):
                v0 = cur[r, pl.ds(c16, 16)]
                v1 = cur[r + 1, pl.ds(c16, 16)]
                v2 = cur[r + 2, pl.ds(c16, 16)]
                v3 = cur[r + 3, pl.ds(c16, 16)]
                am0 = jnp.maximum(am0, v0)
                am1 = jnp.maximum(am1, v1)
                am2 = jnp.maximum(am2, v2)
                am3 = jnp.maximum(am3, v3)
                as0 = as0 + v0
                as1 = as1 + v1
                as2 = as2 + v2
                as3 = as3 + v3
            am = jnp.maximum(jnp.maximum(am0, am1), jnp.maximum(am2, am3))
            asum = (as0 + as1) + (as2 + as3)
            accm[pl.ds(c16, 16)] = jnp.maximum(accm[pl.ds(c16, 16)], am)
            accs[pl.ds(c16, 16)] = accs[pl.ds(c16, 16)] + asum
            return carry
        lax.fori_loop(0, CCH, body, 0)

    pltpu.sync_copy(accm, pmax_hbm.at[wslot, b_local, :])
    pltpu.sync_copy(accs, psum_hbm.at[wslot, b_local, :])


@jax.jit
def kernel(x, W0, b0, W1, b1):
    xt = jnp.transpose(x, (0, 2, 3, 1)).reshape(B, HW, C)

    sc_fn = functools.partial(
        pl.kernel,
        out_type=[
            jax.ShapeDtypeStruct((W_PER_B, B_SC, C), jnp.float32),
            jax.ShapeDtypeStruct((W_PER_B, B_SC, C), jnp.float32),
        ],
        mesh=plsc.VectorSubcoreMesh(core_axis_name="c", subcore_axis_name="s"),
        scratch_types=[
            pltpu.VMEM((HBLK, C), jnp.float32),
            pltpu.VMEM((HBLK, C), jnp.float32),
            pltpu.VMEM((C,), jnp.float32),
            pltpu.VMEM((C,), jnp.float32),
            pltpu.SemaphoreType.DMA,
            pltpu.SemaphoreType.DMA,
        ],
    )(_sc_pool)
    pmax, psum = sc_fn(xt)

    ptc = pl.pallas_call(
        _pool_kernel,
        grid=(NSTEPS,),
        in_specs=[pl.BlockSpec((BB, HW, C), lambda j: (j, 0, 0))],
        out_specs=pl.BlockSpec((BB, C), lambda j: (j, 0)),
        out_shape=jax.ShapeDtypeStruct((B_TC, C), jnp.float32),
    )(xt)

    gate_t = pl.pallas_call(
        _route_kernel,
        in_specs=[
            pl.BlockSpec((B_TC, C), lambda: (0, 0)),
            pl.BlockSpec((W_PER_B, B_SC, C), lambda: (0, 0, 0)),
            pl.BlockSpec((W_PER_B, B_SC, C), lambda: (0, 0, 0)),
            pl.BlockSpec((E, C), lambda: (0, 0)),
            pl.BlockSpec((1, E), lambda: (0, 0)),
            pl.BlockSpec((E, C), lambda: (0, 0)),
            pl.BlockSpec((1, E), lambda: (0, 0)),
        ],
        out_specs=pl.BlockSpec((E, B), lambda: (0, 0)),
        out_shape=jax.ShapeDtypeStruct((E, B), jnp.float32),
    )(ptc, pmax, psum, W0, b0.reshape(1, E), W1, b1.reshape(1, E))
    return gate_t.T


# speedup vs baseline: 1.5358x; 1.5358x over previous
import jax
import jax.numpy as jnp
from jax.experimental import pallas as pl
from jax.experimental.pallas import tpu as pltpu

B, C, H, W = 64, 768, 24, 24
HW = H * W
E = 8
BB = 8
NSTEPS = B // BB
KS = 2
HK = HW // KS
NEG_INF = float("-inf")


def _gate_kernel(x_ref, w0_ref, b0_ref, w1_ref, b1_ref, out_ref,
                 accm, accs):
    j = pl.program_id(0)
    k = pl.program_id(1)
    blk = x_ref[...]
    pmax = jnp.max(blk, axis=1)
    psum = jnp.sum(blk, axis=1)
    rows = pl.ds(j * BB, BB)

    @pl.when(k == 0)
    def _first():
        accm[rows, :] = pmax
        accs[rows, :] = psum

    @pl.when(k > 0)
    def _rest():
        accm[rows, :] = jnp.maximum(accm[rows, :], pmax)
        accs[rows, :] = accs[rows, :] + psum

    @pl.when((j == NSTEPS - 1) & (k == KS - 1))
    def _epilogue():
        pooled = accm[...] + accs[...] * (1.0 / HW)
        dn = (((1,), (1,)), ((), ()))
        h = jax.lax.dot_general(
            pooled, w0_ref[...], dn,
            preferred_element_type=jnp.float32) + b0_ref[...]
        h = jnp.where(h >= 0.0, h, 0.2 * h)
        z = jax.lax.dot_general(
            pooled, w1_ref[...], dn,
            preferred_element_type=jnp.float32) + b1_ref[...]
        noise = jnp.maximum(z, 0.0) + jnp.log1p(jnp.exp(-jnp.abs(z)))
        nmean = jnp.mean(noise, axis=1, keepdims=True)
        var = jnp.sum((noise - nmean) ** 2, axis=1, keepdims=True) / (E - 1)
        norm_noise = (noise - nmean) * jax.lax.rsqrt(var)
        scores = h + norm_noise
        ii = jax.lax.broadcasted_iota(jnp.int32, (B, E), 1)
        m1 = jnp.max(scores, axis=1, keepdims=True)
        i1 = jnp.min(jnp.where(scores == m1, ii, E), axis=1, keepdims=True)
        oh1 = ii == i1
        s2 = jnp.where(oh1, NEG_INF, scores)
        m2 = jnp.max(s2, axis=1, keepdims=True)
        i2 = jnp.min(jnp.where(s2 == m2, ii, E), axis=1, keepdims=True)
        mask = oh1 | (ii == i2)
        hm = jnp.where(mask, h, NEG_INF)
        mx = jnp.max(hm, axis=1, keepdims=True)
        e = jnp.where(mask, jnp.exp(h - mx), 0.0)
        gate = e / jnp.sum(e, axis=1, keepdims=True)
        out_ref[...] = gate.T


@jax.jit
def kernel(x, W0, b0, W1, b1):
    xt = jnp.transpose(x, (0, 2, 3, 1)).reshape(B, HW, C)
    gate_t = pl.pallas_call(
        _gate_kernel,
        grid=(NSTEPS, KS),
        in_specs=[
            pl.BlockSpec((BB, HK, C), lambda j, k: (j, k, 0)),
            pl.BlockSpec((E, C), lambda j, k: (0, 0)),
            pl.BlockSpec((1, E), lambda j, k: (0, 0)),
            pl.BlockSpec((E, C), lambda j, k: (0, 0)),
            pl.BlockSpec((1, E), lambda j, k: (0, 0)),
        ],
        out_specs=pl.BlockSpec((E, B), lambda j, k: (0, 0)),
        out_shape=jax.ShapeDtypeStruct((E, B), jnp.float32),
        scratch_shapes=[
            pltpu.VMEM((B, C), jnp.float32),
            pltpu.VMEM((B, C), jnp.float32),
        ],
    )(xt, W0, b0.reshape(1, E), W1, b1.reshape(1, E))
    return gate_t.T
